# Initial kernel scaffold; baseline (speedup 1.0000x reference)
import jax
import jax.numpy as jnp
from jax.experimental import pallas as pl


def kernel(x, state, edge_index, edge_weight, W_in, b_in, W1, b1, W2, b2):
    raise NotImplementedError("write your pallas kernel here")


# speedup vs baseline: 5.1616x; 5.1616x over previous
import functools

import jax
import jax.numpy as jnp
from jax import lax
from jax.experimental import pallas as pl
from jax.experimental.pallas import tpu as pltpu
from jax.experimental.pallas import tpu_sc as plsc

_EPS = 1e-6
_W_MAX = 5.0
_RSQRT2 = 0.7071067811865476

_NC = 2
_NS = 16
_NW = _NC * _NS
_CHUNK = 80


def _proj_body(D, x_ref, st_ref, winT_ref, bin_ref, w1T_ref, b1_ref,
               w2T_ref, b2_ref, xp_ref, f_ref):
    winT = winT_ref[...]
    xp = (jnp.dot(x_ref[...], winT[:D], preferred_element_type=jnp.float32)
          + jnp.dot(st_ref[...], winT[D:], preferred_element_type=jnp.float32)
          + bin_ref[...])
    xp_ref[...] = xp
    h = jnp.dot(xp, w1T_ref[...], preferred_element_type=jnp.float32) + b1_ref[...]
    h = jnp.maximum(h, 0.1 * h)
    s = jnp.dot(h, w2T_ref[...], preferred_element_type=jnp.float32) + b2_ref[...]
    sig = 1.0 / (1.0 + jnp.exp(-s))
    z = 4.0 * (sig - 0.5)
    f_ref[...] = jnp.log1p(jnp.exp(z))


def _node_proj(x, state, W_in, b_in, W1, b1, W2, b2):
    N, D = x.shape
    BN = 1000
    grid = (N // BN,)
    xp, f = pl.pallas_call(
        functools.partial(_proj_body, D),
        grid=grid,
        in_specs=[
            pl.BlockSpec((BN, D), lambda i: (i, 0)),
            pl.BlockSpec((BN, D), lambda i: (i, 0)),
            pl.BlockSpec((2 * D, D), lambda i: (0, 0)),
            pl.BlockSpec((1, D), lambda i: (0, 0)),
            pl.BlockSpec((D, 16), lambda i: (0, 0)),
            pl.BlockSpec((1, 16), lambda i: (0, 0)),
            pl.BlockSpec((16, 1), lambda i: (0, 0)),
            pl.BlockSpec((1, 1), lambda i: (0, 0)),
        ],
        out_specs=[
            pl.BlockSpec((BN, D), lambda i: (i, 0)),
            pl.BlockSpec((BN, 1), lambda i: (i, 0)),
        ],
        out_shape=[
            jax.ShapeDtypeStruct((N, D), jnp.float32),
            jax.ShapeDtypeStruct((N, 1), jnp.float32),
        ],
    )(x, state, W_in.T, b_in.reshape(1, D), W1.T, b1.reshape(1, 16),
      W2.T, b2.reshape(1, 1))
    return xp, f.reshape(N)


def _sc_body(N, D, nch, xp_hbm, f_hbm, edges_hbm,
             acc_hbm, deg_hbm,
             acc_sh, deg_sh, f_v, e_v, rows_v, wrow_v, sem):
    c = lax.axis_index("c")
    s = lax.axis_index("s")
    wg = c * _NS + s
    nblk = N // _CHUNK

    z16 = jnp.zeros((16,), jnp.float32)

    def _zb(r, carry):
        for j in range(D // 16):
            rows_v[r, pl.ds(j * 16, 16)] = z16
        wrow_v[r] = z16
        return carry
    lax.fori_loop(0, _CHUNK, _zb, 0)

    def _zc(b, carry):
        @pl.when(b % _NS == s)
        def _():
            pltpu.sync_copy(rows_v, acc_sh.at[pl.ds(b * _CHUNK, _CHUNK)])
            pltpu.sync_copy(wrow_v, deg_sh.at[pl.ds(b * _CHUNK, _CHUNK)])
        return carry
    lax.fori_loop(0, nblk, _zc, 0)

    pltpu.sync_copy(f_hbm, f_v)

    plsc.subcore_barrier()

    iot = lax.iota(jnp.int32, 16)
    zi16 = jnp.zeros((16,), jnp.int32)

    def _chunk(k, carry):
        pltpu.sync_copy(edges_hbm.at[wg, k], e_v)
        pltpu.async_copy(xp_hbm.at[e_v.at[0]], rows_v, sem).wait()
        for g in range(_CHUNK // 16):
            sl = pl.ds(g * 16, 16)
            fv = plsc.load_gather(f_v, [e_v[0, sl]])
            wv = plsc.bitcast(e_v[2, sl], jnp.float32) * fv
            wv = jnp.minimum(jnp.maximum(wv, 0.0), _W_MAX)
            plsc.store_scatter(wrow_v, [g * 16 + iot, zi16], wv)
            for l in range(16):
                ws = wv[l]
                r = g * 16 + l
                for j in range(D // 16):
                    sj = pl.ds(j * 16, 16)
                    rows_v[r, sj] = rows_v[r, sj] * ws

        pltpu.sync_copy(rows_v, acc_sh.at[e_v.at[1]], add=True)
        pltpu.sync_copy(wrow_v, deg_sh.at[e_v.at[1]], add=True)
        return carry
    lax.fori_loop(0, nch, _chunk, 0)

    plsc.subcore_barrier()

    def _out(b, carry):
        @pl.when(b % _NS == s)
        def _():
            r0 = b * _CHUNK
            pltpu.sync_copy(acc_sh.at[pl.ds(r0, _CHUNK)], rows_v)
            pltpu.sync_copy(rows_v, acc_hbm.at[c, pl.ds(r0, _CHUNK)])
            pltpu.sync_copy(deg_sh.at[pl.ds(r0, _CHUNK)], wrow_v)
            pltpu.sync_copy(wrow_v, deg_hbm.at[c, pl.ds(r0, _CHUNK)])
        return carry
    lax.fori_loop(0, nblk, _out, 0)


def _sc_aggregate(xp, f, edges):
    N, D = xp.shape
    nch = edges.shape[1]
    mesh = plsc.VectorSubcoreMesh(core_axis_name="c", subcore_axis_name="s",
                                  num_cores=_NC, num_subcores=_NS)
    acc, deg = pl.kernel(
        functools.partial(_sc_body, N, D, nch),
        out_type=(
            jax.ShapeDtypeStruct((_NC, N, D), jnp.float32),
            jax.ShapeDtypeStruct((_NC, N, 16), jnp.float32),
        ),
        mesh=mesh,
        compiler_params=pltpu.CompilerParams(needs_layout_passes=False,
                                             use_tc_tiling_on_sc=False),
        scratch_types=[
            pltpu.VMEM_SHARED((N, D), jnp.float32),
            pltpu.VMEM_SHARED((N, 16), jnp.float32),
            pltpu.VMEM((N,), jnp.float32),
            pltpu.VMEM((3, _CHUNK), jnp.int32),
            pltpu.VMEM((_CHUNK, D), jnp.float32),
            pltpu.VMEM((_CHUNK, 16), jnp.float32),
            pltpu.SemaphoreType.DMA,
        ],
    )(xp, f, edges)
    return acc, deg


def _fin_body(acc_ref, deg_ref, xp_ref, o_ref):
    a = acc_ref[0] + acc_ref[1]
    dg = jnp.sum(deg_ref[0] + deg_ref[1], axis=1)
    o = a / (dg[:, None] + _EPS) + xp_ref[...]
    o_ref[...] = o * 0.5 * (1.0 + lax.erf(o * _RSQRT2))


def _finalize(acc, deg, xp):
    N, D = xp.shape
    BN = 1000
    return pl.pallas_call(
        _fin_body,
        grid=(N // BN,),
        in_specs=[
            pl.BlockSpec((_NC, BN, D), lambda i: (0, i, 0)),
            pl.BlockSpec((_NC, BN, 16), lambda i: (0, i, 0)),
            pl.BlockSpec((BN, D), lambda i: (i, 0)),
        ],
        out_specs=pl.BlockSpec((BN, D), lambda i: (i, 0)),
        out_shape=jax.ShapeDtypeStruct((N, D), jnp.float32),
    )(acc, deg, xp)


def kernel(x, state, edge_index, edge_weight, W_in, b_in, W1, b1, W2, b2):
    N, D = x.shape
    E = edge_weight.shape[0]
    ew_per_w = E // _NW
    nch = ew_per_w // _CHUNK

    src = edge_index[0].astype(jnp.int32).reshape(_NW, nch, _CHUNK)
    dst = edge_index[1].astype(jnp.int32).reshape(_NW, nch, _CHUNK)
    ewb = lax.bitcast_convert_type(edge_weight, jnp.int32).reshape(_NW, nch, _CHUNK)
    edges = jnp.stack([src, dst, ewb], axis=2)

    xp, f = _node_proj(x, state, W_in, b_in, W1, b1, W2, b2)
    acc, deg = _sc_aggregate(xp, f, edges)
    return _finalize(acc, deg, xp)
